# baseline (speedup 1.0000x reference)
import functools

import jax
import jax.numpy as jnp
from jax import lax
from jax.experimental import pallas as pl
from jax.experimental.pallas import tpu as pltpu
from jax.experimental.pallas import tpu_sc as plsc

T = 2048
H = 1024
I = 4096
E = 8
K = 2
P = T * K
B = 128
NB = P // B + E
N_PAD = NB * B
IT = 512
NI = I // IT
NC = 2
NS = 16
NW = NC * NS
L = 16
TRASH_ROW = N_PAD
POS_SZ = P + 8
TRASH_POS = P
SLOT_W = N_PAD // NW
QW = P // 4

_SQRT_HALF = 0.7071067811865476


def _gelu_exact(x):
    return 0.5 * x * (1.0 + lax.erf(x * _SQRT_HALF))


def _router_body(x_ref, rw_ref, ids_ref, w_ref):
    logits = lax.dot_general(
        x_ref[...], rw_ref[...], (((1,), (1,)), ((), ())),
        preferred_element_type=jnp.float32)
    m = jnp.max(logits, axis=-1, keepdims=True)
    ex = jnp.exp(logits - m)
    p = ex / jnp.sum(ex, axis=-1, keepdims=True)
    eidx = lax.broadcasted_iota(jnp.int32, p.shape, 1)
    big = jnp.int32(E + 1)
    m1 = jnp.max(p, axis=-1, keepdims=True)
    a1 = jnp.min(jnp.where(p == m1, eidx, big), axis=-1, keepdims=True)
    p2 = jnp.where(eidx == a1, -jnp.inf, p)
    m2 = jnp.max(p2, axis=-1, keepdims=True)
    a2 = jnp.min(jnp.where(p2 == m2, eidx, big), axis=-1, keepdims=True)
    ids_ref[...] = jnp.concatenate([a1, a2], axis=1)
    w_ref[...] = jnp.concatenate([m1, m2], axis=1)


def _router(x, rw):
    return pl.pallas_call(
        _router_body,
        in_specs=[pl.BlockSpec((T, H), lambda: (0, 0)),
                  pl.BlockSpec((E, H), lambda: (0, 0))],
        out_specs=[pl.BlockSpec((T, K), lambda: (0, 0)),
                   pl.BlockSpec((T, K), lambda: (0, 0))],
        out_shape=[jax.ShapeDtypeStruct((T, K), jnp.int32),
                   jax.ShapeDtypeStruct((T, K), jnp.float32)],
    )(x, rw)


def _dispatch_body(ids_hbm, w_hbm,
                   rows_hbm, wslot_hbm, pos_hbm, bexp_hbm,
                   ida, wq, hist,
                   dest_idx, tok_val, w_val, pos_idx, pos_val,
                   pad_idx, zi, zf, bexp_v, sem):
    wid = lax.axis_index("s") * NC + lax.axis_index("c")
    e = wid >> 2
    q = wid & 3
    iota = lax.broadcasted_iota(jnp.int32, (L,), 0)

    pltpu.sync_copy(ids_hbm, ida)
    pltpu.sync_copy(w_hbm.at[pl.ds(q * QW, QW)], wq)
    hist[...] = jnp.zeros((L,), jnp.int32)
    ones = jnp.ones((L,), jnp.int32)
    qacc = jnp.zeros((L,), jnp.int32)
    qlim = q * QW
    for c in range(P // L):
        idv = ida[pl.ds(c * L, L)]
        plsc.addupdate_scatter(hist, [idv], ones)
        before = jnp.where(jnp.int32(c * L) < qlim, 1, 0)
        qacc = qacc + jnp.where(idv == e, before, 0)
    qpref = jnp.sum(qacc)
    c8 = hist[...]
    padded8 = ((c8 + (B - 1)) >> 7) << 7
    ends8 = jnp.cumsum(padded8)
    base8 = ends8 - padded8
    base_e = jnp.sum(jnp.where(iota == e, base8, 0))
    end_e = jnp.sum(jnp.where(iota == e, ends8, 0))
    cnt_e = jnp.sum(jnp.where(iota == e, c8, 0))
    ends7 = jnp.sum(jnp.where(iota == E - 1, ends8, 0))

    for v in range(B // L):
        zi[pl.ds(v * L, L)] = jnp.zeros((L,), jnp.int32)
        zf[pl.ds(v * L, L)] = jnp.zeros((L,), jnp.float32)

    @pl.when(wid == 0)
    def _():
        for gv in range(3):
            gb = (lax.broadcasted_iota(jnp.int32, (L,), 0) + gv * L) * B
            a = jnp.zeros((L,), jnp.int32)
            for ee in range(E):
                end_s = jnp.sum(jnp.where(iota == ee, ends8, 0))
                a = a + jnp.where(gb >= end_s, 1, 0)
            bexp_v[pl.ds(gv * L, L)] = jnp.minimum(a, E - 1)
        pltpu.sync_copy(bexp_v, bexp_hbm)

    handles = []
    run = base_e + qpref
    for b in range(8):
        for s in range(8):
            c = b * 8 + s
            idv = ida[pl.ds(q * QW + c * L, L)]
            wv = wq[pl.ds(c * L, L)]
            mask = idv == e
            mi = jnp.where(mask, 1, 0)
            rk = jnp.cumsum(mi)
            dest = run + rk - 1
            dest_m = jnp.where(mask, dest, TRASH_ROW)
            dest_m = jnp.clip(dest_m, 0, TRASH_ROW)
            pair = q * QW + c * L + iota
            tok = pair >> 1
            dest_idx[b, pl.ds(s * L, L)] = dest_m
            tok_val[b, pl.ds(s * L, L)] = tok
            w_val[b, pl.ds(s * L, L)] = wv
            pos_idx[b, pl.ds(s * L, L)] = jnp.where(mask, pair, TRASH_POS)
            pos_val[b, pl.ds(s * L, L)] = jnp.clip(dest, 0, N_PAD - 1)
            run = run + jnp.sum(mi)
        handles.append(pltpu.async_copy(
            tok_val.at[b], rows_hbm.at[dest_idx.at[b]], sem))
        handles.append(pltpu.async_copy(
            w_val.at[b], wslot_hbm.at[dest_idx.at[b]], sem))
        handles.append(pltpu.async_copy(
            pos_val.at[b], pos_hbm.at[pos_idx.at[b]], sem))

    @pl.when(q == 3)
    def _():
        pstart = base_e + cnt_e
        for k in range(B // L):
            pidv = pstart + k * L + iota
            pad_idx[0, pl.ds(k * L, L)] = jnp.where(
                pidv < end_e, jnp.clip(pidv, 0, TRASH_ROW), TRASH_ROW)
        pltpu.async_copy(zi, rows_hbm.at[pad_idx.at[0]], sem).wait()
        pltpu.async_copy(zf, wslot_hbm.at[pad_idx.at[0]], sem).wait()

    @pl.when(wid == NW - 1)
    def _():
        for k in range(E):
            tstart = pl.multiple_of(ends7 + k * B, B)
            @pl.when(ends7 + k * B < N_PAD)
            def _():
                pltpu.sync_copy(zi, rows_hbm.at[pl.ds(tstart, B)])
                pltpu.sync_copy(zf, wslot_hbm.at[pl.ds(tstart, B)])

    for h in handles:
        h.wait()


def _sc_mesh():
    return plsc.VectorSubcoreMesh(
        core_axis_name="c", subcore_axis_name="s",
        num_cores=NC, num_subcores=NS)


def _dispatch(ids_flat, w_flat):
    fn = pl.kernel(
        _dispatch_body,
        out_type=[
            jax.ShapeDtypeStruct((N_PAD + L,), jnp.int32),
            jax.ShapeDtypeStruct((N_PAD + L,), jnp.float32),
            jax.ShapeDtypeStruct((POS_SZ,), jnp.int32),
            jax.ShapeDtypeStruct((48,), jnp.int32),
        ],
        mesh=_sc_mesh(),
        scratch_types=[
            pltpu.VMEM((P,), jnp.int32),
            pltpu.VMEM((QW,), jnp.float32),
            pltpu.VMEM((L,), jnp.int32),
            pltpu.VMEM((8, B), jnp.int32),
            pltpu.VMEM((8, B), jnp.int32),
            pltpu.VMEM((8, B), jnp.float32),
            pltpu.VMEM((8, B), jnp.int32),
            pltpu.VMEM((8, B), jnp.int32),
            pltpu.VMEM((1, B), jnp.int32),
            pltpu.VMEM((B,), jnp.int32),
            pltpu.VMEM((B,), jnp.float32),
            pltpu.VMEM((48,), jnp.int32),
            pltpu.SemaphoreType.DMA,
        ],
        compiler_params=pltpu.CompilerParams(needs_layout_passes=False),
    )
    return fn(ids_flat, w_flat)


def _gather_body(x_hbm, rows_hbm, xs_hbm, ridx, xr, sem):
    wid = lax.axis_index("s") * NC + lax.axis_index("c")
    for c2 in range(2):
        start = wid * SLOT_W + c2 * (SLOT_W // 2)
        pltpu.sync_copy(rows_hbm.at[pl.ds(start, SLOT_W // 2)], ridx)
        for v in range(SLOT_W // 2 // L):
            sl = pl.ds(v * L, L)
            ridx[sl] = jnp.clip(ridx[sl], 0, T - 1)
        pltpu.async_copy(x_hbm.at[ridx], xr, sem).wait()
        pltpu.sync_copy(xr, xs_hbm.at[pl.ds(start, SLOT_W // 2)])


def _gather_x(x, rows):
    fn = pl.kernel(
        _gather_body,
        out_type=jax.ShapeDtypeStruct((N_PAD, H), jnp.float32),
        mesh=_sc_mesh(),
        scratch_types=[
            pltpu.VMEM((SLOT_W // 2,), jnp.int32),
            pltpu.VMEM((SLOT_W // 2, H), jnp.float32),
            pltpu.SemaphoreType.DMA,
        ],
    )
    return fn(x, rows)


def _mb_body(bexp_ref, xs_ref, w1_ref, w2_ref, wc_ref, out_ref):
    i = pl.program_id(0)
    g = pl.program_id(1)
    h = lax.dot_general(
        xs_ref[...], w1_ref[0], (((1,), (1,)), ((), ())),
        preferred_element_type=jnp.float32)
    a = _gelu_exact(h)
    part = lax.dot_general(
        a, w2_ref[0], (((1,), (1,)), ((), ())),
        preferred_element_type=jnp.float32)
    part = part * wc_ref[...]
    row0 = pl.multiple_of(g * B, B)

    @pl.when(i == 0)
    def _():
        out_ref[pl.ds(row0, B), :] = part

    @pl.when(i != 0)
    def _():
        out_ref[pl.ds(row0, B), :] += part


def _megablox(bexp, xs, w1, w2, wcol):
    grid_spec = pltpu.PrefetchScalarGridSpec(
        num_scalar_prefetch=1,
        grid=(NI, NB),
        in_specs=[
            pl.BlockSpec((B, H), lambda i, g, b: (g, 0)),
            pl.BlockSpec((1, IT, H), lambda i, g, b: (b[g], i, 0)),
            pl.BlockSpec((1, H, IT), lambda i, g, b: (b[g], 0, i)),
            pl.BlockSpec((B, 1), lambda i, g, b: (g, 0)),
        ],
        out_specs=pl.BlockSpec((N_PAD, H), lambda i, g, b: (0, 0)),
    )
    return pl.pallas_call(
        _mb_body,
        grid_spec=grid_spec,
        out_shape=jax.ShapeDtypeStruct((N_PAD, H), jnp.float32),
    )(bexp, xs, w1, w2, wcol)


def _combine_body(ys_hbm, pos_hbm, bias_hbm, out_hbm,
                  pidx, yr, outv, bias_v, sem):
    wid = lax.axis_index("s") * NC + lax.axis_index("c")
    tpw = T // NW
    pltpu.sync_copy(bias_hbm, bias_v)
    for sc in range(tpw // 16):
        tt = wid * tpw + sc * 16
        pltpu.sync_copy(pos_hbm.at[pl.ds(2 * tt, 32)], pidx)
        for v in range(2):
            sl = pl.ds(v * L, L)
            pidx[sl] = jnp.clip(pidx[sl], 0, N_PAD - 1)
        pltpu.async_copy(ys_hbm.at[pidx], yr, sem).wait()

        def body(i, _):
            for v in range(H // L):
                sl = pl.ds(v * L, L)
                outv[i, sl] = yr[2 * i, sl] + yr[2 * i + 1, sl] + bias_v[sl]
            return 0

        lax.fori_loop(0, 16, body, 0)
        pltpu.sync_copy(outv, out_hbm.at[pl.ds(tt, 16)])


def _combine(ys, pos, bias):
    mesh = plsc.VectorSubcoreMesh(
        core_axis_name="c", subcore_axis_name="s",
        num_cores=NC, num_subcores=NS)
    fn = pl.kernel(
        _combine_body,
        out_type=jax.ShapeDtypeStruct((T, H), jnp.float32),
        mesh=mesh,
        scratch_types=[
            pltpu.VMEM((32,), jnp.int32),
            pltpu.VMEM((32, H), jnp.float32),
            pltpu.VMEM((16, H), jnp.float32),
            pltpu.VMEM((H,), jnp.float32),
            pltpu.SemaphoreType.DMA,
        ],
    )
    return fn(ys, pos, bias)


def kernel(hidden_states, router_w, w1, w2, bias):
    ids2, wt2 = _router(hidden_states, router_w)
    ids_flat = ids2.reshape(P)
    w_flat = wt2.reshape(P)
    rows, wslot, pos, bexp48 = _dispatch(ids_flat, w_flat)
    xs = _gather_x(hidden_states, rows)
    wcol = wslot[:N_PAD].reshape(N_PAD, 1)
    bexp = bexp48[:NB]
    ys = _megablox(bexp, xs, w1, w2, wcol)
    return _combine(ys, pos, bias)


# speedup vs baseline: 4.5120x; 4.5120x over previous
import functools

import jax
import jax.numpy as jnp
from jax import lax
from jax.experimental import pallas as pl
from jax.experimental.pallas import tpu as pltpu
from jax.experimental.pallas import tpu_sc as plsc

T = 2048
H = 1024
I = 4096
E = 8
K = 2
P = T * K
B = 128
NB = P // B + E
N_PAD = NB * B
IT = 512
NI = I // IT
NC = 2
NS = 16
NW = NC * NS
L = 16
NW_TMP = 32
ROWS_SZ = N_PAD + NW_TMP * B
POS_SZ = P + NW_TMP * B
SLOT_W = N_PAD // NW
QW = P // 4

_SQRT_HALF = 0.7071067811865476


def _gelu_exact(x):
    return 0.5 * x * (1.0 + lax.erf(x * _SQRT_HALF))


def _router_body(x_ref, rw_ref, ids_ref, w_ref):
    logits = lax.dot_general(
        x_ref[...], rw_ref[...], (((1,), (1,)), ((), ())),
        preferred_element_type=jnp.float32)
    m = jnp.max(logits, axis=-1, keepdims=True)
    ex = jnp.exp(logits - m)
    p = ex / jnp.sum(ex, axis=-1, keepdims=True)
    eidx = lax.broadcasted_iota(jnp.int32, p.shape, 1)
    big = jnp.int32(E + 1)
    m1 = jnp.max(p, axis=-1, keepdims=True)
    a1 = jnp.min(jnp.where(p == m1, eidx, big), axis=-1, keepdims=True)
    p2 = jnp.where(eidx == a1, -jnp.inf, p)
    m2 = jnp.max(p2, axis=-1, keepdims=True)
    a2 = jnp.min(jnp.where(p2 == m2, eidx, big), axis=-1, keepdims=True)
    ids_ref[...] = jnp.concatenate([a1, a2], axis=1)
    w_ref[...] = jnp.concatenate([m1, m2], axis=1)


def _router(x, rw):
    return pl.pallas_call(
        _router_body,
        in_specs=[pl.BlockSpec((T, H), lambda: (0, 0)),
                  pl.BlockSpec((E, H), lambda: (0, 0))],
        out_specs=[pl.BlockSpec((T, K), lambda: (0, 0)),
                   pl.BlockSpec((T, K), lambda: (0, 0))],
        out_shape=[jax.ShapeDtypeStruct((T, K), jnp.int32),
                   jax.ShapeDtypeStruct((T, K), jnp.float32)],
    )(x, rw)


def _dispatch_body(ids_hbm, w_hbm,
                   rows_hbm, wslot_hbm, pos_hbm, bexp_hbm,
                   ida, wq, hist,
                   dest_idx, tok_val, w_val, pos_idx, pos_val,
                   pad_idx, zi, zf, bexp_v, sem):
    wid = lax.axis_index("s") * NC + lax.axis_index("c")
    e = wid >> 2
    q = wid & 3
    iota = lax.broadcasted_iota(jnp.int32, (L,), 0)

    pltpu.sync_copy(ids_hbm, ida)
    pltpu.sync_copy(w_hbm.at[pl.ds(q * QW, QW)], wq)
    hist[...] = jnp.zeros((L,), jnp.int32)
    ones = jnp.ones((L,), jnp.int32)
    qacc = jnp.zeros((L,), jnp.int32)
    qlim = q * QW
    for c in range(P // L):
        idv = ida[pl.ds(c * L, L)]
        plsc.addupdate_scatter(hist, [idv], ones)
        before = jnp.where(jnp.int32(c * L) < qlim, 1, 0)
        qacc = qacc + jnp.where(idv == e, before, 0)
    qpref = jnp.sum(qacc)
    c8 = hist[...]
    padded8 = ((c8 + (B - 1)) >> 7) << 7
    ends8 = jnp.cumsum(padded8)
    base8 = ends8 - padded8
    base_e = jnp.sum(jnp.where(iota == e, base8, 0))
    end_e = jnp.sum(jnp.where(iota == e, ends8, 0))
    cnt_e = jnp.sum(jnp.where(iota == e, c8, 0))
    ends7 = jnp.sum(jnp.where(iota == E - 1, ends8, 0))

    for v in range(B // L):
        zi[pl.ds(v * L, L)] = jnp.zeros((L,), jnp.int32)
        zf[pl.ds(v * L, L)] = jnp.zeros((L,), jnp.float32)

    @pl.when(wid == 0)
    def _():
        for gv in range(3):
            gb = (lax.broadcasted_iota(jnp.int32, (L,), 0) + gv * L) * B
            a = jnp.zeros((L,), jnp.int32)
            for ee in range(E):
                end_s = jnp.sum(jnp.where(iota == ee, ends8, 0))
                a = a + jnp.where(gb >= end_s, 1, 0)
            bexp_v[pl.ds(gv * L, L)] = jnp.minimum(a, E - 1)
        pltpu.sync_copy(bexp_v, bexp_hbm)

    handles = []
    run = base_e + qpref
    trash_r = N_PAD + wid * B
    trash_p = P + wid * B
    for b in range(8):
        for s in range(8):
            c = b * 8 + s
            lane_tr = s * L + iota
            idv = ida[pl.ds(q * QW + c * L, L)]
            wv = wq[pl.ds(c * L, L)]
            mask = idv == e
            mi = jnp.where(mask, 1, 0)
            rk = jnp.cumsum(mi)
            dest = run + rk - 1
            dest_m = jnp.where(mask, dest, trash_r + lane_tr)
            dest_m = jnp.clip(dest_m, 0, ROWS_SZ - 1)
            pair = q * QW + c * L + iota
            tok = pair >> 1
            dest_idx[b, pl.ds(s * L, L)] = dest_m
            tok_val[b, pl.ds(s * L, L)] = tok
            w_val[b, pl.ds(s * L, L)] = wv
            pos_idx[b, pl.ds(s * L, L)] = jnp.where(
                mask, pair, trash_p + lane_tr)
            pos_val[b, pl.ds(s * L, L)] = jnp.clip(dest, 0, N_PAD - 1)
            run = run + jnp.sum(mi)
        handles.append(pltpu.async_copy(
            tok_val.at[b], rows_hbm.at[dest_idx.at[b]], sem))
        handles.append(pltpu.async_copy(
            w_val.at[b], wslot_hbm.at[dest_idx.at[b]], sem))
        handles.append(pltpu.async_copy(
            pos_val.at[b], pos_hbm.at[pos_idx.at[b]], sem))

    @pl.when(q == 3)
    def _():
        pstart = base_e + cnt_e
        for k in range(B // L):
            pidv = pstart + k * L + iota
            pad_idx[0, pl.ds(k * L, L)] = jnp.where(
                pidv < end_e, jnp.clip(pidv, 0, ROWS_SZ - 1),
                trash_r + k * L + iota)
        pltpu.async_copy(zi, rows_hbm.at[pad_idx.at[0]], sem).wait()
        pltpu.async_copy(zf, wslot_hbm.at[pad_idx.at[0]], sem).wait()

    @pl.when(wid == NW - 1)
    def _():
        for k in range(E):
            tstart = pl.multiple_of(ends7 + k * B, B)
            @pl.when(ends7 + k * B < N_PAD)
            def _():
                pltpu.sync_copy(zi, rows_hbm.at[pl.ds(tstart, B)])
                pltpu.sync_copy(zf, wslot_hbm.at[pl.ds(tstart, B)])

    for h in handles:
        h.wait()


def _sc_mesh():
    return plsc.VectorSubcoreMesh(
        core_axis_name="c", subcore_axis_name="s",
        num_cores=NC, num_subcores=NS)


def _dispatch(ids_flat, w_flat):
    fn = pl.kernel(
        _dispatch_body,
        out_type=[
            jax.ShapeDtypeStruct((ROWS_SZ,), jnp.int32),
            jax.ShapeDtypeStruct((ROWS_SZ,), jnp.float32),
            jax.ShapeDtypeStruct((POS_SZ,), jnp.int32),
            jax.ShapeDtypeStruct((48,), jnp.int32),
        ],
        mesh=_sc_mesh(),
        scratch_types=[
            pltpu.VMEM((P,), jnp.int32),
            pltpu.VMEM((QW,), jnp.float32),
            pltpu.VMEM((L,), jnp.int32),
            pltpu.VMEM((8, B), jnp.int32),
            pltpu.VMEM((8, B), jnp.int32),
            pltpu.VMEM((8, B), jnp.float32),
            pltpu.VMEM((8, B), jnp.int32),
            pltpu.VMEM((8, B), jnp.int32),
            pltpu.VMEM((1, B), jnp.int32),
            pltpu.VMEM((B,), jnp.int32),
            pltpu.VMEM((B,), jnp.float32),
            pltpu.VMEM((48,), jnp.int32),
            pltpu.SemaphoreType.DMA,
        ],
        compiler_params=pltpu.CompilerParams(needs_layout_passes=False),
    )
    return fn(ids_flat, w_flat)


def _gather_body(x_hbm, rows_hbm, xs_hbm, ridx, xr, sem):
    wid = lax.axis_index("s") * NC + lax.axis_index("c")
    for c2 in range(2):
        start = wid * SLOT_W + c2 * (SLOT_W // 2)
        pltpu.sync_copy(rows_hbm.at[pl.ds(start, SLOT_W // 2)], ridx)
        for v in range(SLOT_W // 2 // L):
            sl = pl.ds(v * L, L)
            ridx[sl] = jnp.clip(ridx[sl], 0, T - 1)
        pltpu.async_copy(x_hbm.at[ridx], xr, sem).wait()
        pltpu.sync_copy(xr, xs_hbm.at[pl.ds(start, SLOT_W // 2)])


def _gather_x(x, rows):
    fn = pl.kernel(
        _gather_body,
        out_type=jax.ShapeDtypeStruct((N_PAD, H), jnp.float32),
        mesh=_sc_mesh(),
        scratch_types=[
            pltpu.VMEM((SLOT_W // 2,), jnp.int32),
            pltpu.VMEM((SLOT_W // 2, H), jnp.float32),
            pltpu.SemaphoreType.DMA,
        ],
    )
    return fn(x, rows)


def _mb_body(bexp_ref, xs_ref, w1_ref, w2_ref, wc_ref, out_ref):
    i = pl.program_id(0)
    g = pl.program_id(1)
    h = lax.dot_general(
        xs_ref[...], w1_ref[0], (((1,), (1,)), ((), ())),
        preferred_element_type=jnp.float32)
    a = _gelu_exact(h)
    part = lax.dot_general(
        a, w2_ref[0], (((1,), (1,)), ((), ())),
        preferred_element_type=jnp.float32)
    part = part * wc_ref[...]
    row0 = pl.multiple_of(g * B, B)

    @pl.when(i == 0)
    def _():
        out_ref[pl.ds(row0, B), :] = part

    @pl.when(i != 0)
    def _():
        out_ref[pl.ds(row0, B), :] += part


def _megablox(bexp, xs, w1, w2, wcol):
    grid_spec = pltpu.PrefetchScalarGridSpec(
        num_scalar_prefetch=1,
        grid=(NI, NB),
        in_specs=[
            pl.BlockSpec((B, H), lambda i, g, b: (g, 0)),
            pl.BlockSpec((1, IT, H), lambda i, g, b: (b[g], i, 0)),
            pl.BlockSpec((1, H, IT), lambda i, g, b: (b[g], 0, i)),
            pl.BlockSpec((B, 1), lambda i, g, b: (g, 0)),
        ],
        out_specs=pl.BlockSpec((N_PAD, H), lambda i, g, b: (0, 0)),
    )
    return pl.pallas_call(
        _mb_body,
        grid_spec=grid_spec,
        out_shape=jax.ShapeDtypeStruct((N_PAD, H), jnp.float32),
    )(bexp, xs, w1, w2, wcol)


def _combine_body(ys_hbm, pos_hbm, bias_hbm, out_hbm,
                  pidx, yr, outv, bias_v, sem):
    wid = lax.axis_index("s") * NC + lax.axis_index("c")
    tpw = T // NW
    pltpu.sync_copy(bias_hbm, bias_v)
    for sc in range(tpw // 16):
        tt = wid * tpw + sc * 16
        pltpu.sync_copy(pos_hbm.at[pl.ds(2 * tt, 32)], pidx)
        for v in range(2):
            sl = pl.ds(v * L, L)
            pidx[sl] = jnp.clip(pidx[sl], 0, N_PAD - 1)
        pltpu.async_copy(ys_hbm.at[pidx], yr, sem).wait()

        def body(i, _):
            for v in range(H // L):
                sl = pl.ds(v * L, L)
                outv[i, sl] = yr[2 * i, sl] + yr[2 * i + 1, sl] + bias_v[sl]
            return 0

        lax.fori_loop(0, 16, body, 0)
        pltpu.sync_copy(outv, out_hbm.at[pl.ds(tt, 16)])


def _combine(ys, pos, bias):
    mesh = plsc.VectorSubcoreMesh(
        core_axis_name="c", subcore_axis_name="s",
        num_cores=NC, num_subcores=NS)
    fn = pl.kernel(
        _combine_body,
        out_type=jax.ShapeDtypeStruct((T, H), jnp.float32),
        mesh=mesh,
        scratch_types=[
            pltpu.VMEM((32,), jnp.int32),
            pltpu.VMEM((32, H), jnp.float32),
            pltpu.VMEM((16, H), jnp.float32),
            pltpu.VMEM((H,), jnp.float32),
            pltpu.SemaphoreType.DMA,
        ],
    )
    return fn(ys, pos, bias)


def kernel(hidden_states, router_w, w1, w2, bias):
    ids2, wt2 = _router(hidden_states, router_w)
    ids_flat = ids2.reshape(P)
    w_flat = wt2.reshape(P)
    rows, wslot, pos, bexp48 = _dispatch(ids_flat, w_flat)
    xs = _gather_x(hidden_states, rows)
    wcol = wslot[:N_PAD].reshape(N_PAD, 1)
    bexp = bexp48[:NB]
    ys = _megablox(bexp, xs, w1, w2, wcol)
    return _combine(ys, pos, bias)


# speedup vs baseline: 8.4458x; 1.8719x over previous
import functools

import jax
import jax.numpy as jnp
from jax import lax
from jax.experimental import pallas as pl
from jax.experimental.pallas import tpu as pltpu
from jax.experimental.pallas import tpu_sc as plsc

T = 2048
H = 1024
I = 4096
E = 8
K = 2
P = T * K
B = 128
NB = P // B + E
N_PAD = NB * B
IT = 512
NI = I // IT
NC = 2
NS = 16
NW = NC * NS
L = 16
NW_TMP = 32
ROWS_SZ = N_PAD + NW_TMP * B
POS_SZ = P + NW_TMP * B
SLOT_W = N_PAD // NW
QW = P // 4

_SQRT_HALF = 0.7071067811865476


def _gelu_exact(x):
    return 0.5 * x * (1.0 + lax.erf(x * _SQRT_HALF))


def _router_body(x_ref, rw_ref, ids_ref, w_ref):
    logits = lax.dot_general(
        x_ref[...], rw_ref[...], (((1,), (1,)), ((), ())),
        preferred_element_type=jnp.float32)
    m = jnp.max(logits, axis=-1, keepdims=True)
    ex = jnp.exp(logits - m)
    p = ex / jnp.sum(ex, axis=-1, keepdims=True)
    eidx = lax.broadcasted_iota(jnp.int32, p.shape, 1)
    big = jnp.int32(E + 1)
    m1 = jnp.max(p, axis=-1, keepdims=True)
    a1 = jnp.min(jnp.where(p == m1, eidx, big), axis=-1, keepdims=True)
    p2 = jnp.where(eidx == a1, -jnp.inf, p)
    m2 = jnp.max(p2, axis=-1, keepdims=True)
    a2 = jnp.min(jnp.where(p2 == m2, eidx, big), axis=-1, keepdims=True)
    ids_ref[...] = jnp.concatenate([a1, a2], axis=1)
    w_ref[...] = jnp.concatenate([m1, m2], axis=1)


def _router(x, rw):
    return pl.pallas_call(
        _router_body,
        in_specs=[pl.BlockSpec((T, H), lambda: (0, 0)),
                  pl.BlockSpec((E, H), lambda: (0, 0))],
        out_specs=[pl.BlockSpec((T, K), lambda: (0, 0)),
                   pl.BlockSpec((T, K), lambda: (0, 0))],
        out_shape=[jax.ShapeDtypeStruct((T, K), jnp.int32),
                   jax.ShapeDtypeStruct((T, K), jnp.float32)],
    )(x, rw)


HW_ = P // 2
SP_ROWS = N_PAD + NS * B
SP_POS = P + NS * B


def _dispatch_body(x_hbm, ids_hbm, w_hbm,
                   wslot_hbm, pos_hbm, bexp_hbm, xs_hbm,
                   rows_sp, wslot_sp, pos_sp,
                   ida, wh, hist,
                   dest_idx, tok_val, w_val, pos_idx, pos_val,
                   pad_idx, zi, zf, bexp_v, ridx, xr, wsl_v, pos_v, sem):
    s = lax.axis_index("s")
    core = lax.axis_index("c")
    e = s >> 1
    hf = s & 1
    iota = lax.broadcasted_iota(jnp.int32, (L,), 0)

    pltpu.sync_copy(ids_hbm, ida)
    pltpu.sync_copy(w_hbm.at[pl.ds(hf * HW_, HW_)], wh)
    hist[...] = jnp.zeros((L,), jnp.int32)
    ones = jnp.ones((L,), jnp.int32)
    hacc = jnp.zeros((L,), jnp.int32)
    hlim = hf * HW_
    for c in range(P // L):
        idv = ida[pl.ds(c * L, L)]
        plsc.addupdate_scatter(hist, [idv], ones)
        before = jnp.where(jnp.int32(c * L) < hlim, 1, 0)
        hacc = hacc + jnp.where(idv == e, before, 0)
    hpref = jnp.sum(hacc)
    c8 = hist[...]
    padded8 = ((c8 + (B - 1)) >> 7) << 7
    ends8 = jnp.cumsum(padded8)
    base8 = ends8 - padded8
    base_e = jnp.sum(jnp.where(iota == e, base8, 0))
    end_e = jnp.sum(jnp.where(iota == e, ends8, 0))
    cnt_e = jnp.sum(jnp.where(iota == e, c8, 0))
    ends7 = jnp.sum(jnp.where(iota == E - 1, ends8, 0))

    for v in range(B // L):
        zi[pl.ds(v * L, L)] = jnp.zeros((L,), jnp.int32)
        zf[pl.ds(v * L, L)] = jnp.zeros((L,), jnp.float32)

    @pl.when((s == 0) & (core == 0))
    def _():
        for gv in range(3):
            gb = (lax.broadcasted_iota(jnp.int32, (L,), 0) + gv * L) * B
            a = jnp.zeros((L,), jnp.int32)
            for ee in range(E):
                end_s = jnp.sum(jnp.where(iota == ee, ends8, 0))
                a = a + jnp.where(gb >= end_s, 1, 0)
            bexp_v[pl.ds(gv * L, L)] = jnp.minimum(a, E - 1)
        pltpu.sync_copy(bexp_v, bexp_hbm)

    handles = []
    run = base_e + hpref
    trash_r = N_PAD + s * B
    trash_p = P + s * B
    for b in range(16):
        for s2 in range(8):
            c = b * 8 + s2
            lane_tr = s2 * L + iota
            idv = ida[pl.ds(hf * HW_ + c * L, L)]
            wv = wh[pl.ds(c * L, L)]
            mask = idv == e
            mi = jnp.where(mask, 1, 0)
            rk = jnp.cumsum(mi)
            dest = run + rk - 1
            dest_m = jnp.where(mask, dest, trash_r + lane_tr)
            dest_m = jnp.clip(dest_m, 0, SP_ROWS - 1)
            pair = hf * HW_ + c * L + iota
            tok = pair >> 1
            dest_idx[b, pl.ds(s2 * L, L)] = dest_m
            tok_val[b, pl.ds(s2 * L, L)] = tok
            w_val[b, pl.ds(s2 * L, L)] = wv
            pos_idx[b, pl.ds(s2 * L, L)] = jnp.where(
                mask, pair, trash_p + lane_tr)
            pos_val[b, pl.ds(s2 * L, L)] = jnp.clip(dest, 0, N_PAD - 1)
            run = run + jnp.sum(mi)
        handles.append(pltpu.async_copy(
            tok_val.at[b], rows_sp.at[dest_idx.at[b]], sem))
        handles.append(pltpu.async_copy(
            w_val.at[b], wslot_sp.at[dest_idx.at[b]], sem))
        handles.append(pltpu.async_copy(
            pos_val.at[b], pos_sp.at[pos_idx.at[b]], sem))

    @pl.when(hf == 1)
    def _():
        pstart = base_e + cnt_e
        for k in range(B // L):
            pidv = pstart + k * L + iota
            pad_idx[0, pl.ds(k * L, L)] = jnp.where(
                pidv < end_e, jnp.clip(pidv, 0, SP_ROWS - 1),
                trash_r + k * L + iota)
        pltpu.async_copy(zi, rows_sp.at[pad_idx.at[0]], sem).wait()
        pltpu.async_copy(zf, wslot_sp.at[pad_idx.at[0]], sem).wait()

    @pl.when(s == NS - 1)
    def _():
        for k in range(E):
            tstart = pl.multiple_of(ends7 + k * B, B)
            @pl.when(ends7 + k * B < N_PAD)
            def _():
                pltpu.sync_copy(zi, rows_sp.at[pl.ds(tstart, B)])
                pltpu.sync_copy(zf, wslot_sp.at[pl.ds(tstart, B)])

    for h in handles:
        h.wait()
    plsc.subcore_barrier()

    w = core * NS + s
    sl_out = pl.ds(w * SLOT_W, SLOT_W)
    pltpu.sync_copy(wslot_sp.at[sl_out], wsl_v)
    pltpu.sync_copy(wsl_v, wslot_hbm.at[sl_out])
    sl_pos = pl.ds(w * (P // NW), P // NW)
    pltpu.sync_copy(pos_sp.at[sl_pos], pos_v)
    pltpu.sync_copy(pos_v, pos_hbm.at[sl_pos])
    for c2 in range(2):
        start = w * SLOT_W + c2 * (SLOT_W // 2)
        pltpu.sync_copy(rows_sp.at[pl.ds(start, SLOT_W // 2)], ridx)
        for v in range(SLOT_W // 2 // L):
            slv = pl.ds(v * L, L)
            ridx[slv] = jnp.clip(ridx[slv], 0, T - 1)
        pltpu.async_copy(x_hbm.at[ridx], xr, sem).wait()
        pltpu.sync_copy(xr, xs_hbm.at[pl.ds(start, SLOT_W // 2)])


def _sc_mesh():
    return plsc.VectorSubcoreMesh(
        core_axis_name="c", subcore_axis_name="s",
        num_cores=NC, num_subcores=NS)


def _dispatch(x, ids_flat, w_flat):
    fn = pl.kernel(
        _dispatch_body,
        out_type=[
            jax.ShapeDtypeStruct((N_PAD,), jnp.float32),
            jax.ShapeDtypeStruct((P,), jnp.int32),
            jax.ShapeDtypeStruct((48,), jnp.int32),
            jax.ShapeDtypeStruct((N_PAD, H), jnp.float32),
        ],
        mesh=_sc_mesh(),
        scratch_types=[
            pltpu.VMEM_SHARED((SP_ROWS,), jnp.int32),
            pltpu.VMEM_SHARED((SP_ROWS,), jnp.float32),
            pltpu.VMEM_SHARED((SP_POS,), jnp.int32),
            pltpu.VMEM((P,), jnp.int32),
            pltpu.VMEM((HW_,), jnp.float32),
            pltpu.VMEM((L,), jnp.int32),
            pltpu.VMEM((16, B), jnp.int32),
            pltpu.VMEM((16, B), jnp.int32),
            pltpu.VMEM((16, B), jnp.float32),
            pltpu.VMEM((16, B), jnp.int32),
            pltpu.VMEM((16, B), jnp.int32),
            pltpu.VMEM((1, B), jnp.int32),
            pltpu.VMEM((B,), jnp.int32),
            pltpu.VMEM((B,), jnp.float32),
            pltpu.VMEM((48,), jnp.int32),
            pltpu.VMEM((SLOT_W // 2,), jnp.int32),
            pltpu.VMEM((SLOT_W // 2, H), jnp.float32),
            pltpu.VMEM((SLOT_W,), jnp.float32),
            pltpu.VMEM((P // NW,), jnp.int32),
            pltpu.SemaphoreType.DMA,
        ],
        compiler_params=pltpu.CompilerParams(needs_layout_passes=False),
    )
    return fn(x, ids_flat, w_flat)


def _mb_body(bexp_ref, xs_ref, w1_ref, w2_ref, wc_ref, out_ref):
    i = pl.program_id(0)
    g = pl.program_id(1)
    h = lax.dot_general(
        xs_ref[...], w1_ref[0], (((1,), (1,)), ((), ())),
        preferred_element_type=jnp.float32)
    a = _gelu_exact(h)
    part = lax.dot_general(
        a, w2_ref[0], (((1,), (1,)), ((), ())),
        preferred_element_type=jnp.float32)
    part = part * wc_ref[...]
    row0 = pl.multiple_of(g * B, B)

    @pl.when(i == 0)
    def _():
        out_ref[pl.ds(row0, B), :] = part

    @pl.when(i != 0)
    def _():
        out_ref[pl.ds(row0, B), :] += part


def _megablox(bexp, xs, w1, w2, wcol):
    grid_spec = pltpu.PrefetchScalarGridSpec(
        num_scalar_prefetch=1,
        grid=(NI, NB),
        in_specs=[
            pl.BlockSpec((B, H), lambda i, g, b: (g, 0)),
            pl.BlockSpec((1, IT, H), lambda i, g, b: (b[g], i, 0)),
            pl.BlockSpec((1, H, IT), lambda i, g, b: (b[g], 0, i)),
            pl.BlockSpec((B, 1), lambda i, g, b: (g, 0)),
        ],
        out_specs=pl.BlockSpec((N_PAD, H), lambda i, g, b: (0, 0)),
    )
    return pl.pallas_call(
        _mb_body,
        grid_spec=grid_spec,
        out_shape=jax.ShapeDtypeStruct((N_PAD, H), jnp.float32),
    )(bexp, xs, w1, w2, wcol)


def _combine_body(ys_hbm, pos_hbm, bias_hbm, out_hbm,
                  pidx, yr, outv, bias_v, sem):
    wid = lax.axis_index("s") * NC + lax.axis_index("c")
    tpw = T // NW
    pltpu.sync_copy(bias_hbm, bias_v)
    for sc in range(tpw // 16):
        tt = wid * tpw + sc * 16
        pltpu.sync_copy(pos_hbm.at[pl.ds(2 * tt, 32)], pidx)
        for v in range(2):
            sl = pl.ds(v * L, L)
            pidx[sl] = jnp.clip(pidx[sl], 0, N_PAD - 1)
        pltpu.async_copy(ys_hbm.at[pidx], yr, sem).wait()

        def body(i, _):
            for v in range(H // L):
                sl = pl.ds(v * L, L)
                outv[i, sl] = yr[2 * i, sl] + yr[2 * i + 1, sl] + bias_v[sl]
            return 0

        lax.fori_loop(0, 16, body, 0)
        pltpu.sync_copy(outv, out_hbm.at[pl.ds(tt, 16)])


def _combine(ys, pos, bias):
    mesh = plsc.VectorSubcoreMesh(
        core_axis_name="c", subcore_axis_name="s",
        num_cores=NC, num_subcores=NS)
    fn = pl.kernel(
        _combine_body,
        out_type=jax.ShapeDtypeStruct((T, H), jnp.float32),
        mesh=mesh,
        scratch_types=[
            pltpu.VMEM((32,), jnp.int32),
            pltpu.VMEM((32, H), jnp.float32),
            pltpu.VMEM((16, H), jnp.float32),
            pltpu.VMEM((H,), jnp.float32),
            pltpu.SemaphoreType.DMA,
        ],
    )
    return fn(ys, pos, bias)


def kernel(hidden_states, router_w, w1, w2, bias):
    ids2, wt2 = _router(hidden_states, router_w)
    ids_flat = ids2.reshape(P)
    w_flat = wt2.reshape(P)
    wslot, pos, bexp48, xs = _dispatch(hidden_states, ids_flat, w_flat)
    wcol = wslot.reshape(N_PAD, 1)
    bexp = bexp48[:NB]
    ys = _megablox(bexp, xs, w1, w2, wcol)
    return _combine(ys, pos, bias)


# speedup vs baseline: 9.4621x; 1.1203x over previous
import functools

import jax
import jax.numpy as jnp
from jax import lax
from jax.experimental import pallas as pl
from jax.experimental.pallas import tpu as pltpu
from jax.experimental.pallas import tpu_sc as plsc

T = 2048
H = 1024
I = 4096
E = 8
K = 2
P = T * K
B = 256
BSH = 8
NB = P // B + E
N_PAD = NB * B
IT = 512
NI = I // IT
NC = 2
NS = 16
NW = NC * NS
L = 16
SW = 128
SLOT_W = N_PAD // NW
GC = 64

_SQRT_HALF = 0.7071067811865476


def _gelu_exact(x):
    return 0.5 * x * (1.0 + lax.erf(x * _SQRT_HALF))


def _router_body(x_ref, rw_ref, ids_ref, w_ref):
    logits = lax.dot_general(
        x_ref[...], rw_ref[...], (((1,), (1,)), ((), ())),
        preferred_element_type=jnp.float32)
    m = jnp.max(logits, axis=-1, keepdims=True)
    ex = jnp.exp(logits - m)
    p = ex / jnp.sum(ex, axis=-1, keepdims=True)
    eidx = lax.broadcasted_iota(jnp.int32, p.shape, 1)
    big = jnp.int32(E + 1)
    m1 = jnp.max(p, axis=-1, keepdims=True)
    a1 = jnp.min(jnp.where(p == m1, eidx, big), axis=-1, keepdims=True)
    p2 = jnp.where(eidx == a1, -jnp.inf, p)
    m2 = jnp.max(p2, axis=-1, keepdims=True)
    a2 = jnp.min(jnp.where(p2 == m2, eidx, big), axis=-1, keepdims=True)
    ids_ref[...] = jnp.concatenate([a1, a2], axis=1)
    w_ref[...] = jnp.concatenate([m1, m2], axis=1)


def _router(x, rw):
    return pl.pallas_call(
        _router_body,
        in_specs=[pl.BlockSpec((T, H), lambda: (0, 0)),
                  pl.BlockSpec((E, H), lambda: (0, 0))],
        out_specs=[pl.BlockSpec((T, K), lambda: (0, 0)),
                   pl.BlockSpec((T, K), lambda: (0, 0))],
        out_shape=[jax.ShapeDtypeStruct((T, K), jnp.int32),
                   jax.ShapeDtypeStruct((T, K), jnp.float32)],
    )(x, rw)


HW_ = P // 2
SP_ROWS = N_PAD + NS * SW
SP_POS = P + NS * SW


def _dispatch_body(x_hbm, ids_hbm, w_hbm,
                   wslot_hbm, pos_hbm, bexp_hbm, xs_hbm,
                   rows_sp, wslot_sp, pos_sp,
                   ida, wh, hist,
                   dest_idx, tok_val, w_val, pos_idx, pos_val,
                   pad_idx, zi, zf, bexp_v, ridx, xr, wsl_v, pos_v, sem):
    s = lax.axis_index("s")
    core = lax.axis_index("c")
    e = s >> 1
    hf = s & 1
    iota = lax.broadcasted_iota(jnp.int32, (L,), 0)

    pltpu.sync_copy(ids_hbm, ida)
    pltpu.sync_copy(w_hbm.at[pl.ds(hf * HW_, HW_)], wh)
    hist[...] = jnp.zeros((L,), jnp.int32)
    ones = jnp.ones((L,), jnp.int32)
    hacc = jnp.zeros((L,), jnp.int32)
    hlim = hf * HW_
    for c in range(P // L):
        idv = ida[pl.ds(c * L, L)]
        plsc.addupdate_scatter(hist, [idv], ones)
        before = jnp.where(jnp.int32(c * L) < hlim, 1, 0)
        hacc = hacc + jnp.where(idv == e, before, 0)
    hpref = jnp.sum(hacc)
    c8 = hist[...]
    padded8 = ((c8 + (B - 1)) >> BSH) << BSH
    ends8 = jnp.cumsum(padded8)
    base8 = ends8 - padded8
    base_e = jnp.sum(jnp.where(iota == e, base8, 0))
    end_e = jnp.sum(jnp.where(iota == e, ends8, 0))
    cnt_e = jnp.sum(jnp.where(iota == e, c8, 0))
    ends7 = jnp.sum(jnp.where(iota == E - 1, ends8, 0))

    for v in range(SW // L):
        zi[pl.ds(v * L, L)] = jnp.zeros((L,), jnp.int32)
        zf[pl.ds(v * L, L)] = jnp.zeros((L,), jnp.float32)

    @pl.when((s == 0) & (core == 0))
    def _():
        for gv in range(3):
            gb = (lax.broadcasted_iota(jnp.int32, (L,), 0) + gv * L) * B
            a = jnp.zeros((L,), jnp.int32)
            for ee in range(E):
                end_s = jnp.sum(jnp.where(iota == ee, ends8, 0))
                a = a + jnp.where(gb >= end_s, 1, 0)
            bexp_v[pl.ds(gv * L, L)] = jnp.minimum(a, E - 1)
        pltpu.sync_copy(bexp_v, bexp_hbm)

    handles = []
    run = base_e + hpref
    trash_r = N_PAD + s * SW
    trash_p = P + s * SW
    for b in range(16):
        for s2 in range(8):
            c = b * 8 + s2
            lane_tr = s2 * L + iota
            idv = ida[pl.ds(hf * HW_ + c * L, L)]
            wv = wh[pl.ds(c * L, L)]
            mask = idv == e
            mi = jnp.where(mask, 1, 0)
            rk = jnp.cumsum(mi)
            dest = run + rk - 1
            dest_m = jnp.where(mask, dest, trash_r + lane_tr)
            dest_m = jnp.clip(dest_m, 0, SP_ROWS - 1)
            pair = hf * HW_ + c * L + iota
            tok = pair >> 1
            dest_idx[b, pl.ds(s2 * L, L)] = dest_m
            tok_val[b, pl.ds(s2 * L, L)] = tok
            w_val[b, pl.ds(s2 * L, L)] = wv
            pos_idx[b, pl.ds(s2 * L, L)] = jnp.where(
                mask, pair, trash_p + lane_tr)
            pos_val[b, pl.ds(s2 * L, L)] = jnp.clip(dest, 0, N_PAD - 1)
            run = run + jnp.sum(mi)
        handles.append(pltpu.async_copy(
            tok_val.at[b], rows_sp.at[dest_idx.at[b]], sem))
        handles.append(pltpu.async_copy(
            w_val.at[b], wslot_sp.at[dest_idx.at[b]], sem))
        handles.append(pltpu.async_copy(
            pos_val.at[b], pos_sp.at[pos_idx.at[b]], sem))

    @pl.when(hf == 1)
    def _():
        pstart = base_e + cnt_e
        for kk in range(B // SW):
            for k in range(SW // L):
                pidv = pstart + kk * SW + k * L + iota
                pad_idx[kk, pl.ds(k * L, L)] = jnp.where(
                    pidv < end_e, jnp.clip(pidv, 0, SP_ROWS - 1),
                    trash_r + k * L + iota)
            pltpu.async_copy(zi, rows_sp.at[pad_idx.at[kk]], sem).wait()
            pltpu.async_copy(zf, wslot_sp.at[pad_idx.at[kk]], sem).wait()

    @pl.when(s == NS - 1)
    def _():
        for k in range(E * B // SW):
            tstart = pl.multiple_of(ends7 + k * SW, SW)
            @pl.when(ends7 + k * SW < N_PAD)
            def _():
                pltpu.sync_copy(zi, rows_sp.at[pl.ds(tstart, SW)])
                pltpu.sync_copy(zf, wslot_sp.at[pl.ds(tstart, SW)])

    for h in handles:
        h.wait()
    plsc.subcore_barrier()

    w = core * NS + s
    sl_out = pl.ds(w * SLOT_W, SLOT_W)
    pltpu.sync_copy(wslot_sp.at[sl_out], wsl_v)
    pltpu.sync_copy(wsl_v, wslot_hbm.at[sl_out])
    sl_pos = pl.ds(w * (P // NW), P // NW)
    pltpu.sync_copy(pos_sp.at[sl_pos], pos_v)
    pltpu.sync_copy(pos_v, pos_hbm.at[sl_pos])
    for c2 in range(SLOT_W // GC):
        start = w * SLOT_W + c2 * GC
        pltpu.sync_copy(rows_sp.at[pl.ds(start, GC)], ridx)
        for v in range(GC // L):
            slv = pl.ds(v * L, L)
            ridx[slv] = jnp.clip(ridx[slv], 0, T - 1)
        pltpu.async_copy(x_hbm.at[ridx], xr, sem).wait()
        pltpu.sync_copy(xr, xs_hbm.at[pl.ds(start, GC)])


def _sc_mesh():
    return plsc.VectorSubcoreMesh(
        core_axis_name="c", subcore_axis_name="s",
        num_cores=NC, num_subcores=NS)


def _dispatch(x, ids_flat, w_flat):
    fn = pl.kernel(
        _dispatch_body,
        out_type=[
            jax.ShapeDtypeStruct((N_PAD,), jnp.float32),
            jax.ShapeDtypeStruct((P,), jnp.int32),
            jax.ShapeDtypeStruct((48,), jnp.int32),
            jax.ShapeDtypeStruct((N_PAD, H), jnp.float32),
        ],
        mesh=_sc_mesh(),
        scratch_types=[
            pltpu.VMEM_SHARED((SP_ROWS,), jnp.int32),
            pltpu.VMEM_SHARED((SP_ROWS,), jnp.float32),
            pltpu.VMEM_SHARED((SP_POS,), jnp.int32),
            pltpu.VMEM((P,), jnp.int32),
            pltpu.VMEM((HW_,), jnp.float32),
            pltpu.VMEM((L,), jnp.int32),
            pltpu.VMEM((16, SW), jnp.int32),
            pltpu.VMEM((16, SW), jnp.int32),
            pltpu.VMEM((16, SW), jnp.float32),
            pltpu.VMEM((16, SW), jnp.int32),
            pltpu.VMEM((16, SW), jnp.int32),
            pltpu.VMEM((B // SW, SW), jnp.int32),
            pltpu.VMEM((SW,), jnp.int32),
            pltpu.VMEM((SW,), jnp.float32),
            pltpu.VMEM((48,), jnp.int32),
            pltpu.VMEM((GC,), jnp.int32),
            pltpu.VMEM((GC, H), jnp.float32),
            pltpu.VMEM((SLOT_W,), jnp.float32),
            pltpu.VMEM((P // NW,), jnp.int32),
            pltpu.SemaphoreType.DMA,
        ],
        compiler_params=pltpu.CompilerParams(needs_layout_passes=False),
    )
    return fn(x, ids_flat, w_flat)


def _mb_body(bexp_ref, xs_ref, w1_ref, w2_ref, wc_ref, out_ref):
    i = pl.program_id(0)
    g = pl.program_id(1)
    h = lax.dot_general(
        xs_ref[...], w1_ref[0], (((1,), (1,)), ((), ())),
        preferred_element_type=jnp.float32)
    a = _gelu_exact(h)
    part = lax.dot_general(
        a, w2_ref[0], (((1,), (1,)), ((), ())),
        preferred_element_type=jnp.float32)
    part = part * wc_ref[...]
    row0 = pl.multiple_of(g * B, B)

    @pl.when(i == 0)
    def _():
        out_ref[pl.ds(row0, B), :] = part

    @pl.when(i != 0)
    def _():
        out_ref[pl.ds(row0, B), :] += part


def _megablox(bexp, xs, w1, w2, wcol):
    grid_spec = pltpu.PrefetchScalarGridSpec(
        num_scalar_prefetch=1,
        grid=(NI, NB),
        in_specs=[
            pl.BlockSpec((B, H), lambda i, g, b: (g, 0)),
            pl.BlockSpec((1, IT, H), lambda i, g, b: (b[g], i, 0)),
            pl.BlockSpec((1, H, IT), lambda i, g, b: (b[g], 0, i)),
            pl.BlockSpec((B, 1), lambda i, g, b: (g, 0)),
        ],
        out_specs=pl.BlockSpec((N_PAD, H), lambda i, g, b: (0, 0)),
    )
    return pl.pallas_call(
        _mb_body,
        grid_spec=grid_spec,
        out_shape=jax.ShapeDtypeStruct((N_PAD, H), jnp.float32),
    )(bexp, xs, w1, w2, wcol)


def _combine_body(ys_hbm, pos_hbm, bias_hbm, out_hbm,
                  pidx, yr, outv, bias_v, sem):
    wid = lax.axis_index("s") * NC + lax.axis_index("c")
    tpw = T // NW
    pltpu.sync_copy(bias_hbm, bias_v)
    for sc in range(tpw // 16):
        tt = wid * tpw + sc * 16
        pltpu.sync_copy(pos_hbm.at[pl.ds(2 * tt, 32)], pidx)
        for v in range(2):
            sl = pl.ds(v * L, L)
            pidx[sl] = jnp.clip(pidx[sl], 0, N_PAD - 1)
        pltpu.async_copy(ys_hbm.at[pidx], yr, sem).wait()

        def body(i, _):
            for v in range(H // L):
                sl = pl.ds(v * L, L)
                outv[i, sl] = yr[2 * i, sl] + yr[2 * i + 1, sl] + bias_v[sl]
            return 0

        lax.fori_loop(0, 16, body, 0)
        pltpu.sync_copy(outv, out_hbm.at[pl.ds(tt, 16)])


def _combine(ys, pos, bias):
    mesh = plsc.VectorSubcoreMesh(
        core_axis_name="c", subcore_axis_name="s",
        num_cores=NC, num_subcores=NS)
    fn = pl.kernel(
        _combine_body,
        out_type=jax.ShapeDtypeStruct((T, H), jnp.float32),
        mesh=mesh,
        scratch_types=[
            pltpu.VMEM((32,), jnp.int32),
            pltpu.VMEM((32, H), jnp.float32),
            pltpu.VMEM((16, H), jnp.float32),
            pltpu.VMEM((H,), jnp.float32),
            pltpu.SemaphoreType.DMA,
        ],
    )
    return fn(ys, pos, bias)


def kernel(hidden_states, router_w, w1, w2, bias):
    ids2, wt2 = _router(hidden_states, router_w)
    ids_flat = ids2.reshape(P)
    w_flat = wt2.reshape(P)
    wslot, pos, bexp48, xs = _dispatch(hidden_states, ids_flat, w_flat)
    wcol = wslot.reshape(N_PAD, 1)
    bexp = bexp48[:NB]
    ys = _megablox(bexp, xs, w1, w2, wcol)
    return _combine(ys, pos, bias)


# speedup vs baseline: 11.5558x; 1.2213x over previous
import functools

import jax
import jax.numpy as jnp
from jax import lax
from jax.experimental import pallas as pl
from jax.experimental.pallas import tpu as pltpu
from jax.experimental.pallas import tpu_sc as plsc

T = 2048
H = 1024
I = 4096
E = 8
K = 2
P = T * K
B = 256
BSH = 8
NB = P // B + E
N_PAD = NB * B
IT = 1024
NI = I // IT
NC = 2
NS = 16
NW = NC * NS
L = 16
SW = 128
SLOT_W = N_PAD // NW
GC = 64

_SQRT_HALF = 0.7071067811865476


def _gelu_exact(x):
    return 0.5 * x * (1.0 + lax.erf(x * _SQRT_HALF))


def _router_body(x_ref, rw_ref, ids_ref, w_ref):
    logits = lax.dot_general(
        x_ref[...], rw_ref[...], (((1,), (1,)), ((), ())),
        preferred_element_type=jnp.float32)
    m = jnp.max(logits, axis=-1, keepdims=True)
    ex = jnp.exp(logits - m)
    p = ex / jnp.sum(ex, axis=-1, keepdims=True)
    eidx = lax.broadcasted_iota(jnp.int32, p.shape, 1)
    big = jnp.int32(E + 1)
    m1 = jnp.max(p, axis=-1, keepdims=True)
    a1 = jnp.min(jnp.where(p == m1, eidx, big), axis=-1, keepdims=True)
    p2 = jnp.where(eidx == a1, -jnp.inf, p)
    m2 = jnp.max(p2, axis=-1, keepdims=True)
    a2 = jnp.min(jnp.where(p2 == m2, eidx, big), axis=-1, keepdims=True)
    ids_ref[...] = jnp.concatenate([a1, a2], axis=1)
    w_ref[...] = jnp.concatenate([m1, m2], axis=1)


def _router(x, rw):
    return pl.pallas_call(
        _router_body,
        in_specs=[pl.BlockSpec((T, H), lambda: (0, 0)),
                  pl.BlockSpec((E, H), lambda: (0, 0))],
        out_specs=[pl.BlockSpec((T, K), lambda: (0, 0)),
                   pl.BlockSpec((T, K), lambda: (0, 0))],
        out_shape=[jax.ShapeDtypeStruct((T, K), jnp.int32),
                   jax.ShapeDtypeStruct((T, K), jnp.float32)],
    )(x, rw)


HW_ = P // 2
SP_ROWS = N_PAD + NS * SW
SP_POS = P + NS * SW


def _dispatch_body(x_hbm, ids_hbm, w_hbm,
                   wslot_hbm, pos_hbm, bexp_hbm, xs_hbm,
                   rows_sp, wslot_sp, pos_sp,
                   ida, wh, hist,
                   dest_idx, tok_val, w_val, pos_idx, pos_val,
                   pad_idx, zi, zf, bexp_v, ridx, xr, wsl_v, pos_v, sem):
    s = lax.axis_index("s")
    core = lax.axis_index("c")
    e = s >> 1
    hf = s & 1
    iota = lax.broadcasted_iota(jnp.int32, (L,), 0)

    pltpu.sync_copy(ids_hbm, ida)
    pltpu.sync_copy(w_hbm.at[pl.ds(hf * HW_, HW_)], wh)
    hist[...] = jnp.zeros((L,), jnp.int32)
    ones = jnp.ones((L,), jnp.int32)
    hacc = jnp.zeros((L,), jnp.int32)
    hlim = hf * HW_
    for c in range(P // L):
        idv = ida[pl.ds(c * L, L)]
        plsc.addupdate_scatter(hist, [idv], ones)
        before = jnp.where(jnp.int32(c * L) < hlim, 1, 0)
        hacc = hacc + jnp.where(idv == e, before, 0)
    hpref = jnp.sum(hacc)
    c8 = hist[...]
    padded8 = ((c8 + (B - 1)) >> BSH) << BSH
    ends8 = jnp.cumsum(padded8)
    base8 = ends8 - padded8
    base_e = jnp.sum(jnp.where(iota == e, base8, 0))
    end_e = jnp.sum(jnp.where(iota == e, ends8, 0))
    cnt_e = jnp.sum(jnp.where(iota == e, c8, 0))
    ends7 = jnp.sum(jnp.where(iota == E - 1, ends8, 0))

    for v in range(SW // L):
        zi[pl.ds(v * L, L)] = jnp.zeros((L,), jnp.int32)
        zf[pl.ds(v * L, L)] = jnp.zeros((L,), jnp.float32)

    @pl.when((s == 0) & (core == 0))
    def _():
        for gv in range(NB // L + 1):
            gb = (lax.broadcasted_iota(jnp.int32, (L,), 0) + gv * L) * B
            a = jnp.zeros((L,), jnp.int32)
            for ee in range(E):
                end_s = jnp.sum(jnp.where(iota == ee, ends8, 0))
                a = a + jnp.where(gb >= end_s, 1, 0)
            bexp_v[pl.ds(gv * L, L)] = jnp.where(
                a >= E, 15, jnp.minimum(a, E - 1))
        pltpu.sync_copy(bexp_v, bexp_hbm)

    handles = []
    run = base_e + hpref
    trash_r = N_PAD + s * SW
    trash_p = P + s * SW
    for b in range(16):
        for s2 in range(8):
            c = b * 8 + s2
            lane_tr = s2 * L + iota
            idv = ida[pl.ds(hf * HW_ + c * L, L)]
            wv = wh[pl.ds(c * L, L)]
            mask = idv == e
            mi = jnp.where(mask, 1, 0)
            rk = jnp.cumsum(mi)
            dest = run + rk - 1
            dest_m = jnp.where(mask, dest, trash_r + lane_tr)
            dest_m = jnp.clip(dest_m, 0, SP_ROWS - 1)
            pair = hf * HW_ + c * L + iota
            tok = pair >> 1
            dest_idx[b, pl.ds(s2 * L, L)] = dest_m
            tok_val[b, pl.ds(s2 * L, L)] = tok
            w_val[b, pl.ds(s2 * L, L)] = wv
            pos_idx[b, pl.ds(s2 * L, L)] = jnp.where(
                mask, pair, trash_p + lane_tr)
            pos_val[b, pl.ds(s2 * L, L)] = jnp.clip(dest, 0, N_PAD - 1)
            run = run + jnp.sum(mi)
        handles.append(pltpu.async_copy(
            tok_val.at[b], rows_sp.at[dest_idx.at[b]], sem))
        handles.append(pltpu.async_copy(
            w_val.at[b], wslot_sp.at[dest_idx.at[b]], sem))
        handles.append(pltpu.async_copy(
            pos_val.at[b], pos_sp.at[pos_idx.at[b]], sem))

    @pl.when(hf == 1)
    def _():
        pstart = base_e + cnt_e
        for kk in range(B // SW):
            for k in range(SW // L):
                pidv = pstart + kk * SW + k * L + iota
                pad_idx[kk, pl.ds(k * L, L)] = jnp.where(
                    pidv < end_e, jnp.clip(pidv, 0, SP_ROWS - 1),
                    trash_r + k * L + iota)
            pltpu.async_copy(zi, rows_sp.at[pad_idx.at[kk]], sem).wait()
            pltpu.async_copy(zf, wslot_sp.at[pad_idx.at[kk]], sem).wait()

    @pl.when(s == NS - 1)
    def _():
        for k in range(E * B // SW):
            tstart = pl.multiple_of(ends7 + k * SW, SW)
            @pl.when(ends7 + k * SW < N_PAD)
            def _():
                pltpu.sync_copy(zi, rows_sp.at[pl.ds(tstart, SW)])
                pltpu.sync_copy(zf, wslot_sp.at[pl.ds(tstart, SW)])

    for h in handles:
        h.wait()
    plsc.subcore_barrier()

    w = core * NS + s
    sl_out = pl.ds(w * SLOT_W, SLOT_W)
    pltpu.sync_copy(wslot_sp.at[sl_out], wsl_v)
    pltpu.sync_copy(wsl_v, wslot_hbm.at[sl_out])
    sl_pos = pl.ds(w * (P // NW), P // NW)
    pltpu.sync_copy(pos_sp.at[sl_pos], pos_v)
    pltpu.sync_copy(pos_v, pos_hbm.at[sl_pos])
    for c2 in range(SLOT_W // GC):
        start = w * SLOT_W + c2 * GC
        pltpu.sync_copy(rows_sp.at[pl.ds(start, GC)], ridx)
        for v in range(GC // L):
            slv = pl.ds(v * L, L)
            ridx[slv] = jnp.clip(ridx[slv], 0, T - 1)
        pltpu.async_copy(x_hbm.at[ridx], xr, sem).wait()
        pltpu.sync_copy(xr, xs_hbm.at[pl.ds(start, GC)])


def _sc_mesh():
    return plsc.VectorSubcoreMesh(
        core_axis_name="c", subcore_axis_name="s",
        num_cores=NC, num_subcores=NS)


def _dispatch(x, ids_flat, w_flat):
    fn = pl.kernel(
        _dispatch_body,
        out_type=[
            jax.ShapeDtypeStruct((N_PAD,), jnp.float32),
            jax.ShapeDtypeStruct((P,), jnp.int32),
            jax.ShapeDtypeStruct((48,), jnp.int32),
            jax.ShapeDtypeStruct((N_PAD, H), jnp.float32),
        ],
        mesh=_sc_mesh(),
        scratch_types=[
            pltpu.VMEM_SHARED((SP_ROWS,), jnp.int32),
            pltpu.VMEM_SHARED((SP_ROWS,), jnp.float32),
            pltpu.VMEM_SHARED((SP_POS,), jnp.int32),
            pltpu.VMEM((P,), jnp.int32),
            pltpu.VMEM((HW_,), jnp.float32),
            pltpu.VMEM((L,), jnp.int32),
            pltpu.VMEM((16, SW), jnp.int32),
            pltpu.VMEM((16, SW), jnp.int32),
            pltpu.VMEM((16, SW), jnp.float32),
            pltpu.VMEM((16, SW), jnp.int32),
            pltpu.VMEM((16, SW), jnp.int32),
            pltpu.VMEM((B // SW, SW), jnp.int32),
            pltpu.VMEM((SW,), jnp.int32),
            pltpu.VMEM((SW,), jnp.float32),
            pltpu.VMEM((48,), jnp.int32),
            pltpu.VMEM((GC,), jnp.int32),
            pltpu.VMEM((GC, H), jnp.float32),
            pltpu.VMEM((SLOT_W,), jnp.float32),
            pltpu.VMEM((P // NW,), jnp.int32),
            pltpu.SemaphoreType.DMA,
        ],
        compiler_params=pltpu.CompilerParams(needs_layout_passes=False),
    )
    return fn(x, ids_flat, w_flat)


def _mb_body(bexp_ref, xs_ref, w1_ref, w2_ref, wc_ref, out_ref):
    i = pl.program_id(0)
    g = pl.program_id(1)

    @pl.when(bexp_ref[g] < E)
    def _():
        h = lax.dot_general(
            xs_ref[...], w1_ref[0], (((1,), (1,)), ((), ())),
            preferred_element_type=jnp.float32)
        a = _gelu_exact(h)
        part = lax.dot_general(
            a, w2_ref[0], (((1,), (1,)), ((), ())),
            preferred_element_type=jnp.float32)
        part = part * wc_ref[...]
        row0 = pl.multiple_of(g * B, B)

        @pl.when(i == 0)
        def _():
            out_ref[pl.ds(row0, B), :] = part

        @pl.when(i != 0)
        def _():
            out_ref[pl.ds(row0, B), :] += part


def _megablox(bexp, xs, w1, w2, wcol):
    grid_spec = pltpu.PrefetchScalarGridSpec(
        num_scalar_prefetch=1,
        grid=(NI, NB),
        in_specs=[
            pl.BlockSpec((B, H), lambda i, g, b: (g, 0)),
            pl.BlockSpec((1, IT, H), lambda i, g, b: (b[g] & 7, i, 0)),
            pl.BlockSpec((1, H, IT), lambda i, g, b: (b[g] & 7, 0, i)),
            pl.BlockSpec((B, 1), lambda i, g, b: (g, 0)),
        ],
        out_specs=pl.BlockSpec((N_PAD, H), lambda i, g, b: (0, 0)),
    )
    return pl.pallas_call(
        _mb_body,
        grid_spec=grid_spec,
        out_shape=jax.ShapeDtypeStruct((N_PAD, H), jnp.float32),
    )(bexp, xs, w1, w2, wcol)


def _combine_body(ys_hbm, pos_hbm, bias_hbm, out_hbm,
                  pidx, yr, outv, bias_v, sem):
    wid = lax.axis_index("s") * NC + lax.axis_index("c")
    tpw = T // NW
    pltpu.sync_copy(bias_hbm, bias_v)
    for sc in range(tpw // 16):
        tt = wid * tpw + sc * 16
        pltpu.sync_copy(pos_hbm.at[pl.ds(2 * tt, 32)], pidx)
        for v in range(2):
            sl = pl.ds(v * L, L)
            pidx[sl] = jnp.clip(pidx[sl], 0, N_PAD - 1)
        pltpu.async_copy(ys_hbm.at[pidx], yr, sem).wait()

        def body(i, _):
            for v in range(H // L):
                sl = pl.ds(v * L, L)
                outv[i, sl] = yr[2 * i, sl] + yr[2 * i + 1, sl] + bias_v[sl]
            return 0

        lax.fori_loop(0, 16, body, 0)
        pltpu.sync_copy(outv, out_hbm.at[pl.ds(tt, 16)])


def _combine(ys, pos, bias):
    mesh = plsc.VectorSubcoreMesh(
        core_axis_name="c", subcore_axis_name="s",
        num_cores=NC, num_subcores=NS)
    fn = pl.kernel(
        _combine_body,
        out_type=jax.ShapeDtypeStruct((T, H), jnp.float32),
        mesh=mesh,
        scratch_types=[
            pltpu.VMEM((32,), jnp.int32),
            pltpu.VMEM((32, H), jnp.float32),
            pltpu.VMEM((16, H), jnp.float32),
            pltpu.VMEM((H,), jnp.float32),
            pltpu.SemaphoreType.DMA,
        ],
    )
    return fn(ys, pos, bias)


def kernel(hidden_states, router_w, w1, w2, bias):
    ids2, wt2 = _router(hidden_states, router_w)
    ids_flat = ids2.reshape(P)
    w_flat = wt2.reshape(P)
    wslot, pos, bexp48, xs = _dispatch(hidden_states, ids_flat, w_flat)
    wcol = wslot.reshape(N_PAD, 1)
    bexp = bexp48[:NB]
    ys = _megablox(bexp, xs, w1, w2, wcol)
    return _combine(ys, pos, bias)
